# Initial kernel scaffold; baseline (speedup 1.0000x reference)
import jax
import jax.numpy as jnp
from jax.experimental import pallas as pl


def kernel(inputs, gamma, beta, emb_table):
    raise NotImplementedError("write your pallas kernel here")


# speedup vs baseline: 2.6995x; 2.6995x over previous
import math

import jax
import jax.numpy as jnp
from jax.experimental import pallas as pl
from jax.experimental.pallas import tpu as pltpu

_LN_EPS = 1e-5


def _body(x_ref, g_ref, b_ref, e_ref, o_ref):
    lb, c = e_ref.shape
    d = c // 2
    x = x_ref[...]
    mean = jnp.mean(x, axis=-1, keepdims=True)
    xc = x - mean
    var = jnp.mean(xc * xc, axis=-1, keepdims=True)
    xn = xc * jax.lax.rsqrt(var + _LN_EPS)
    xn = xn * g_ref[...] + b_ref[...]

    i = pl.program_id(0)
    pos = (i * lb + jax.lax.broadcasted_iota(jnp.int32, (lb, 1), 0)).astype(jnp.float32)
    dep = jax.lax.broadcasted_iota(jnp.int32, (1, d), 1).astype(jnp.float32) * jnp.float32(1.0 / d)
    rate = jnp.exp(dep * jnp.float32(-math.log(10000.0)))
    ang = pos * rate
    pe = jnp.concatenate([jnp.sin(ang), jnp.cos(ang)], axis=-1)

    add = e_ref[...] * 20.0 + pe
    o_ref[...] = xn + add[None, :, :]


def kernel(inputs, gamma, beta, emb_table):
    b, l, c = inputs.shape
    lb = 512
    grid = (l // lb,)
    gamma2 = gamma.reshape(1, c)
    beta2 = beta.reshape(1, c)
    return pl.pallas_call(
        _body,
        grid=grid,
        in_specs=[
            pl.BlockSpec((b, lb, c), lambda i: (0, i, 0)),
            pl.BlockSpec((1, c), lambda i: (0, 0)),
            pl.BlockSpec((1, c), lambda i: (0, 0)),
            pl.BlockSpec((lb, c), lambda i: (i, 0)),
        ],
        out_specs=pl.BlockSpec((b, lb, c), lambda i: (0, i, 0)),
        out_shape=jax.ShapeDtypeStruct((b, l, c), inputs.dtype),
        compiler_params=pltpu.CompilerParams(
            dimension_semantics=("arbitrary",),
        ),
    )(inputs, gamma2, beta2, emb_table)
